# baseline (speedup 1.0000x reference)
import functools

import jax
import jax.numpy as jnp
from jax import lax
from jax.experimental import pallas as pl
from jax.experimental.pallas import tpu as pltpu
from jax.experimental.pallas import tpu_sc as plsc

B = 16384
V = 1000000
D = 64

NC, NS = 2, 16
NW = NC * NS
BPW = B // NW
ICH = 128
NCH = BPW // ICH


def _sc_gather_body(uid_hbm, iid_hbm, ut_hbm, it_hbm, e1_hbm, e2_hbm,
                    uidx, iidx, urows, irows, sem):
    wid = lax.axis_index("s") * NC + lax.axis_index("c")
    base = wid * BPW
    pltpu.sync_copy(uid_hbm.at[wid], uidx)
    pltpu.sync_copy(iid_hbm.at[wid], iidx)
    copies = []
    for j in range(NCH):
        copies.append(pltpu.async_copy(
            ut_hbm.at[uidx.at[j]], urows.at[pl.ds(j * ICH, ICH)], sem))
        copies.append(pltpu.async_copy(
            it_hbm.at[iidx.at[j]], irows.at[pl.ds(j * ICH, ICH)], sem))
    for c in copies:
        c.wait()
    pltpu.sync_copy(urows, e1_hbm.at[pl.ds(base, BPW)])
    pltpu.sync_copy(irows, e2_hbm.at[pl.ds(base, BPW)])


@functools.lru_cache(maxsize=None)
def _sc_gather():
    return pl.kernel(
        _sc_gather_body,
        out_type=(
            jax.ShapeDtypeStruct((B, D), jnp.float32),
            jax.ShapeDtypeStruct((B, D), jnp.float32),
        ),
        mesh=plsc.VectorSubcoreMesh(core_axis_name="c", subcore_axis_name="s"),
        scratch_types=[
            pltpu.VMEM((NCH, ICH), jnp.int32),
            pltpu.VMEM((NCH, ICH), jnp.int32),
            pltpu.VMEM((BPW, D), jnp.float32),
            pltpu.VMEM((BPW, D), jnp.float32),
            pltpu.SemaphoreType.DMA,
        ],
        compiler_params=pltpu.CompilerParams(use_tc_tiling_on_sc=False),
    )


BLK = 2048


def _mlp_body(e1_ref, e2_ref, w1a_ref, w1b_ref, b1_ref, w2_ref, b2_ref,
              w3_ref, b3_ref, out_ref):
    h = jnp.dot(e1_ref[...], w1a_ref[...], preferred_element_type=jnp.float32)
    h += jnp.dot(e2_ref[...], w1b_ref[...], preferred_element_type=jnp.float32)
    h = jnp.maximum(h + b1_ref[...], 0.0)
    h = jnp.maximum(
        jnp.dot(h, w2_ref[...], preferred_element_type=jnp.float32)
        + b2_ref[...], 0.0)
    out_ref[...] = jnp.maximum(
        jnp.dot(h, w3_ref[...], preferred_element_type=jnp.float32)
        + b3_ref[...], 0.0)


def _full(shape):
    return pl.BlockSpec(shape, lambda i: (0,) * len(shape))


@functools.lru_cache(maxsize=None)
def _mlp():
    return pl.pallas_call(
        _mlp_body,
        grid=(B // BLK,),
        in_specs=[
            pl.BlockSpec((BLK, D), lambda i: (i, 0)),
            pl.BlockSpec((BLK, D), lambda i: (i, 0)),
            _full((D, 256)),
            _full((D, 256)),
            _full((1, 256)),
            _full((256, 128)),
            _full((1, 128)),
            _full((128, 64)),
            _full((1, 64)),
        ],
        out_specs=pl.BlockSpec((BLK, 64), lambda i: (i, 0)),
        out_shape=jax.ShapeDtypeStruct((B, 64), jnp.float32),
    )


def kernel(user_id, item_id, emb_user, emb_item, W1, b1, W2, b2, W3, b3):
    uid = user_id.astype(jnp.int32).reshape(NW, NCH, ICH)
    iid = item_id.astype(jnp.int32).reshape(NW, NCH, ICH)
    e1, e2 = _sc_gather()(uid, iid, emb_user, emb_item)
    w1a = W1[:, :D].T
    w1b = W1[:, D:].T
    return _mlp()(e1, e2, w1a, w1b, b1[None, :], W2.T, b2[None, :],
                  W3.T, b3[None, :])
